# baseline (speedup 1.0000x reference)
import functools

import jax
import jax.numpy as jnp
from jax import lax
from jax.experimental import pallas as pl
from jax.experimental.pallas import tpu as pltpu
from jax.experimental.pallas import tpu_sc as plsc

_NCLASS = 19
_B = 4
_C = 128
_HW = 128
_HWUP = 512
_NCHUNK = 4
_CSZ = _C // _NCHUNK
_NSUB = 16
_QH_PER = _HW // _NSUB
_NROW = 48


_ACC_N = _NCLASS * _QH_PER * _HW


def _md_body(masks_hbm, md_hbm, mrows, acc, qw0_s, qw1_s, ww0_s, ww1_s):
    c = lax.axis_index("c")
    t = lax.axis_index("s")
    iota = lax.iota(jnp.int32, 16)
    zv = jnp.zeros((16,), jnp.float32)
    onev = jnp.ones((16,), jnp.float32)

    def build_tab(g, _):
        iw = iota * 32 + g
        qw0 = lax.shift_right_arithmetic(2 * iw - 3, 3)
        qw1 = qw0 + 1
        f = iw.astype(jnp.float32) * 0.25 - 0.375 - qw0.astype(jnp.float32)
        zerov = jnp.zeros((16,), jnp.float32)
        ww0 = jnp.where(qw0 < 0, zerov,
                        jnp.where(qw1 > (_HW - 1), onev, 1.0 - f))
        ww1 = jnp.where(qw1 > (_HW - 1), zerov,
                        jnp.where(qw0 < 0, onev, f))
        qw0_s[pl.ds(g * 16, 16)] = lax.max(qw0, 0)
        qw1_s[pl.ds(g * 16, 16)] = lax.min(qw1, _HW - 1)
        ww0_s[pl.ds(g * 16, 16)] = ww0
        ww1_s[pl.ds(g * 16, 16)] = ww1
        return 0

    lax.fori_loop(0, 32, build_tab, 0)

    for i in range(2):
        img = c * 2 + i

        def zero_acc(z, _):
            acc[pl.ds(z * 16, 16)] = zv
            return 0

        lax.fori_loop(0, _ACC_N // 16, zero_acc, 0)

        row_lo = lax.clamp(0, 32 * t - 8, _HWUP - _NROW)
        src_off = pl.multiple_of(img * _HWUP * _HWUP + row_lo * _HWUP, 8)
        pltpu.sync_copy(masks_hbm.at[pl.ds(src_off, _NROW * _HWUP)], mrows)
        r_start = lax.max(32 * t - 2, 0) - row_lo
        r_end = lax.min(32 * t + 34, _HWUP) - row_lo

        def do_row(r, _):
            ih = row_lo + r
            q0 = lax.shift_right_arithmetic(2 * ih - 3, 3)
            q1 = q0 + 1
            d0 = q0 - _QH_PER * t
            d1 = d0 + 1
            fv = (jnp.full((16,), ih, jnp.int32).astype(jnp.float32) * 0.25
                  - 0.375 - jnp.full((16,), q0, jnp.int32).astype(jnp.float32))
            zrv = jnp.zeros((16,), jnp.float32)
            wh0 = jnp.where(jnp.full((16,), q1 > (_HW - 1)), onev, 1.0 - fv)
            wh1 = jnp.where(jnp.full((16,), q0 < 0), onev, fv)
            wh0 = jnp.where(jnp.full((16,), (d0 >= 0) & (d0 < _QH_PER)),
                            wh0, zrv)
            wh1 = jnp.where(jnp.full((16,), (d1 >= 0) & (d1 < _QH_PER)),
                            wh1, zrv)
            b0 = jnp.full((16,), lax.clamp(0, d0, _QH_PER - 1) * _HW, jnp.int32)
            b1 = jnp.full((16,), lax.clamp(0, d1, _QH_PER - 1) * _HW, jnp.int32)
            rbase = r * _HWUP
            for g in range(32):
                kv = plsc.load_gather(mrows, [iota * 32 + (rbase + g)])
                k10 = lax.shift_left(kv, 10)
                qw0 = qw0_s[pl.ds(g * 16, 16)]
                qw1 = qw1_s[pl.ds(g * 16, 16)]
                ww0 = ww0_s[pl.ds(g * 16, 16)]
                ww1 = ww1_s[pl.ds(g * 16, 16)]
                plsc.addupdate_scatter(acc, [k10 + (b0 + qw0)], wh0 * ww0)
                plsc.addupdate_scatter(acc, [k10 + (b0 + qw1)], wh0 * ww1)
                plsc.addupdate_scatter(acc, [k10 + (b1 + qw0)], wh1 * ww0)
                plsc.addupdate_scatter(acc, [k10 + (b1 + qw1)], wh1 * ww1)
            return 0

        lax.fori_loop(r_start, r_end, do_row, 0)
        off = pl.multiple_of((img * _NSUB + t) * _ACC_N, 8)
        pltpu.sync_copy(acc, md_hbm.at[pl.ds(off, _ACC_N)])


_md_sc = functools.partial(
    pl.kernel,
    out_type=jax.ShapeDtypeStruct((_B * _NSUB * _ACC_N,), jnp.float32),
    mesh=plsc.VectorSubcoreMesh(core_axis_name="c", subcore_axis_name="s"),
    compiler_params=pltpu.CompilerParams(needs_layout_passes=False),
    scratch_types=[
        pltpu.VMEM((_NROW * _HWUP,), jnp.int32),
        pltpu.VMEM((_ACC_N,), jnp.float32),
        pltpu.VMEM((_HWUP,), jnp.int32),
        pltpu.VMEM((_HWUP,), jnp.int32),
        pltpu.VMEM((_HWUP,), jnp.float32),
        pltpu.VMEM((_HWUP,), jnp.float32),
    ],
)(_md_body)


def _feats_copy(feats_hbm, fv_ref, sem, img, slot, ch):
    return pltpu.make_async_copy(
        feats_hbm.at[img, pl.ds(ch * _CSZ, _CSZ)],
        fv_ref.at[slot, ch],
        sem.at[slot, ch],
    )


def _proto_body(md_ref, feats_hbm, out_ref, fv_ref, sem):
    b = pl.program_id(0)
    slot = jax.lax.rem(b, 2)

    @pl.when(b == 0)
    def _():
        for ch in range(_NCHUNK):
            _feats_copy(feats_hbm, fv_ref, sem, 0, 0, ch).start()

    @pl.when(b < _B - 1)
    def _():
        for ch in range(_NCHUNK):
            _feats_copy(feats_hbm, fv_ref, sem, b + 1, 1 - slot, ch).start()

    md_blk = md_ref[0]
    cnts = jnp.zeros((_NCLASS,), jnp.float32)
    for t in range(_NSUB):
        cnts = cnts + jnp.sum(md_blk[t], axis=1)
    scale = (1.0 / _B) / (cnts[:, None] + 1e-6)

    for ch in range(_NCHUNK):
        _feats_copy(feats_hbm, fv_ref, sem, b, slot, ch).wait()
    fv_img = fv_ref[slot].reshape(_C, _HW * _HW)
    sums = jnp.zeros((_NCLASS, _C), jnp.float32)
    for t in range(_NSUB):
        fv_t = fv_img[:, t * 1024:(t + 1) * 1024]
        sums = sums + jax.lax.dot_general(
            md_blk[t], fv_t, (((1,), (1,)), ((), ())),
            preferred_element_type=jnp.float32)
    contrib = sums * scale

    @pl.when(b == 0)
    def _():
        out_ref[...] = contrib

    @pl.when(b > 0)
    def _():
        out_ref[...] += contrib


@jax.jit
def kernel(feats, masks):
    feats_flat = feats.reshape(_B, _C, _HW * _HW)
    md = _md_sc(masks.reshape(-1)).reshape(_B, _NSUB, _NCLASS, _QH_PER * _HW)
    out = pl.pallas_call(
        _proto_body,
        grid=(_B,),
        in_specs=[
            pl.BlockSpec((1, _NSUB, _NCLASS, _QH_PER * _HW),
                         lambda b: (b, 0, 0, 0)),
            pl.BlockSpec(memory_space=pl.ANY),
        ],
        out_specs=pl.BlockSpec((_NCLASS, _C), lambda b: (0, 0)),
        out_shape=jax.ShapeDtypeStruct((_NCLASS, _C), jnp.float32),
        scratch_shapes=[
            pltpu.VMEM((2, _NCHUNK, _CSZ, _HW * _HW), jnp.float32),
            pltpu.SemaphoreType.DMA((2, _NCHUNK)),
        ],
        compiler_params=pltpu.CompilerParams(
            dimension_semantics=("arbitrary",),
        ),
    )(md, feats_flat)
    return out


# speedup vs baseline: 1.9785x; 1.9785x over previous
import jax
import jax.numpy as jnp
import numpy as np
from jax.experimental import pallas as pl
from jax.experimental.pallas import tpu as pltpu

_NCLASS = 19
_CPAD = 24
_B = 4
_C = 128
_HW = 128
_HWUP = 512
_NCHUNK = 8
_CSZ = _C // _NCHUNK
_NMAP = 5


def _resize_weight_mat(in_size, out_size):
    scale = out_size / in_size
    sample_f = (np.arange(out_size) + 0.5) / scale - 0.5
    x = np.abs(sample_f[None, :] - np.arange(in_size)[:, None])
    w = np.maximum(0.0, 1.0 - x)
    tot = w.sum(axis=0, keepdims=True)
    w = np.where(np.abs(tot) > 1e-6, w / tot, 0.0)
    keep = (sample_f >= -0.5) & (sample_f <= in_size - 0.5)
    return np.where(keep[None, :], w, 0.0).T.astype(np.float32)


_W_NP = _resize_weight_mat(_HW, _HWUP)


def _feats_copy(feats_hbm, fv_ref, sem, img, slot, c):
    return pltpu.make_async_copy(
        feats_hbm.at[img, pl.ds(c * _CSZ, _CSZ)],
        fv_ref.at[slot, c],
        sem.at[slot, c],
    )


def _proto_body(mask_ref, w_ref, feats_hbm, out_ref, md_ref, fv_ref, sem):
    b = pl.program_id(0)
    slot = jax.lax.rem(b, 2)

    @pl.when(b == 0)
    def _():
        for c in range(_NCHUNK):
            _feats_copy(feats_hbm, fv_ref, sem, 0, 0, c).start()
        md_ref[_NCLASS:] = jnp.zeros((_CPAD - _NCLASS, _HW * _HW), jnp.float32)

    @pl.when(b < _B - 1)
    def _():
        for c in range(_NCHUNK):
            _feats_copy(feats_hbm, fv_ref, sem, b + 1, 1 - slot, c).start()

    m16 = mask_ref[0].astype(jnp.int16)
    w = w_ref[...]

    for j in range(_NMAP):
        k0 = 4 * j
        kn = min(4, _NCLASS - k0)
        eqc = jnp.zeros((_HWUP, _HWUP), jnp.bfloat16)
        for i in reversed(range(kn)):
            eqc = jnp.where(m16 == jnp.int16(k0 + i),
                            jnp.bfloat16(float(64 ** i)), eqc)
        t1c = jax.lax.dot_general(eqc, w, (((1,), (0,)), ((), ())),
                                  preferred_element_type=jnp.float32)
        u = t1c * (8.0 / float(64 ** (kn - 1)))
        for i in reversed(range(kn)):
            if i > 0:
                d = jnp.floor(u)
                u = (u - d) * 64.0
            else:
                d = u
            t1k = (d * 0.125).astype(jnp.bfloat16)
            md = jax.lax.dot_general(w, t1k, (((0,), (0,)), ((), ())),
                                     preferred_element_type=jnp.float32)
            md_ref[k0 + i] = jnp.reshape(md, (_HW * _HW,))

    md_all = md_ref[...]
    cnts = jnp.sum(md_all, axis=1)
    scale = (1.0 / _B) / (cnts[:_NCLASS, None] + 1e-6)

    for c in range(_NCHUNK):
        _feats_copy(feats_hbm, fv_ref, sem, b, slot, c).wait()
    fv_img = fv_ref[slot].reshape(_C, _HW * _HW)
    sums = jax.lax.dot_general(md_all, fv_img, (((1,), (1,)), ((), ())),
                               preferred_element_type=jnp.float32)
    contrib = sums[:_NCLASS] * scale

    @pl.when(b == 0)
    def _():
        out_ref[...] = contrib

    @pl.when(b > 0)
    def _():
        out_ref[...] += contrib


@jax.jit
def kernel(feats, masks):
    feats_flat = feats.reshape(_B, _C, _HW * _HW)
    w_bf = jnp.asarray(_W_NP, jnp.bfloat16)
    out = pl.pallas_call(
        _proto_body,
        grid=(_B,),
        in_specs=[
            pl.BlockSpec((1, _HWUP, _HWUP), lambda b: (b, 0, 0)),
            pl.BlockSpec((_HWUP, _HW), lambda b: (0, 0)),
            pl.BlockSpec(memory_space=pl.ANY),
        ],
        out_specs=pl.BlockSpec((_NCLASS, _C), lambda b: (0, 0)),
        out_shape=jax.ShapeDtypeStruct((_NCLASS, _C), jnp.float32),
        scratch_shapes=[
            pltpu.VMEM((_CPAD, _HW * _HW), jnp.float32),
            pltpu.VMEM((2, _NCHUNK, _CSZ, _HW * _HW), jnp.float32),
            pltpu.SemaphoreType.DMA((2, _NCHUNK)),
        ],
        compiler_params=pltpu.CompilerParams(
            dimension_semantics=("arbitrary",),
        ),
    )(masks, w_bf, feats_flat)
    return out
